# baseline (speedup 1.0000x reference)
import functools

import jax
import jax.numpy as jnp
from jax import lax
from jax.experimental import pallas as pl
from jax.experimental.pallas import tpu as pltpu
from jax.experimental.pallas import tpu_sc as plsc

_V = 1000000
_B = 4096
_L = 200
_D = 64
_C = 20
_NC = 2
_NS = 16
_NW = _NC * _NS
_SPW = _B // _NW
_H0 = 96
_H1 = _L - _H0


def _make_gather_pool():
    mesh = plsc.VectorSubcoreMesh(
        core_axis_name="c", subcore_axis_name="s",
        num_cores=_NC, num_subcores=_NS,
    )

    @functools.partial(
        pl.kernel,
        out_type=jax.ShapeDtypeStruct((_B, _D), jnp.float32),
        mesh=mesh,
        name="gather_pool",
        compiler_params=pltpu.CompilerParams(use_tc_tiling_on_sc=False),
        scratch_types=[
            pltpu.VMEM((_SPW, _L), jnp.int32),
            pltpu.VMEM((_L, _D), jnp.float32),
            pltpu.VMEM((_L, _D), jnp.float32),
            pltpu.VMEM((_SPW, _D), jnp.float32),
            pltpu.SemaphoreType.DMA,
            pltpu.SemaphoreType.DMA,
        ],
    )
    def gather_pool(x_hbm, table_hbm, out_hbm,
                    idx_v, rows_a, rows_b, pooled_v, sem_a, sem_b):
        wid = lax.axis_index("s") * _NC + lax.axis_index("c")
        pltpu.sync_copy(x_hbm.at[pl.ds(wid * _SPW, _SPW)], idx_v)

        def fire(i, rows, sem):
            pltpu.async_copy(table_hbm.at[idx_v.at[i, pl.ds(0, _H0)]],
                             rows.at[pl.ds(0, _H0)], sem)
            pltpu.async_copy(table_hbm.at[idx_v.at[i, pl.ds(_H0, _H1)]],
                             rows.at[pl.ds(_H0, _H1)], sem)

        def wait(i, rows, sem):
            pltpu.make_async_copy(table_hbm.at[idx_v.at[i, pl.ds(0, _H0)]],
                                  rows.at[pl.ds(0, _H0)], sem).wait()
            pltpu.make_async_copy(table_hbm.at[idx_v.at[i, pl.ds(_H0, _H1)]],
                                  rows.at[pl.ds(_H0, _H1)], sem).wait()

        def consume(i, rows):
            def body(l, accs):
                return tuple(a + rows[l, pl.ds(q * 16, 16)]
                             for q, a in enumerate(accs))
            accs = tuple(jnp.zeros((16,), jnp.float32) for _ in range(4))
            accs = lax.fori_loop(0, _L, body, accs, unroll=4)
            for q in range(4):
                pooled_v[i, pl.ds(q * 16, 16)] = accs[q]

        fire(0, rows_a, sem_a)

        def step(k, carry):
            g = 2 * k
            fire(g + 1, rows_b, sem_b)
            wait(g, rows_a, sem_a)
            consume(g, rows_a)

            @pl.when(g + 2 < _SPW)
            def _():
                fire(g + 2, rows_a, sem_a)

            wait(g + 1, rows_b, sem_b)
            consume(g + 1, rows_b)
            return carry

        lax.fori_loop(0, _SPW // 2, step, 0)
        pltpu.sync_copy(pooled_v, out_hbm.at[pl.ds(wid * _SPW, _SPW)])

    return gather_pool


_gather_pool = _make_gather_pool()


_CH = 160
_NCH = _V // _CH
_KMAX = (_NCH + _NW - 1) // _NW


def _make_depad_pack():
    mesh = plsc.VectorSubcoreMesh(
        core_axis_name="c", subcore_axis_name="s",
        num_cores=_NC, num_subcores=_NS,
    )

    @functools.partial(
        pl.kernel,
        out_type=jax.ShapeDtypeStruct((_V // 2, 2 * _D), jnp.float32),
        mesh=mesh,
        name="depad_pack",
        compiler_params=pltpu.CompilerParams(use_tc_tiling_on_sc=True),
        scratch_types=[
            pltpu.VMEM((2, _CH, _D), jnp.float32),
            pltpu.VMEM((2, _CH // 2, 2 * _D), jnp.float32),
            pltpu.SemaphoreType.DMA,
            pltpu.SemaphoreType.DMA,
            pltpu.SemaphoreType.DMA,
            pltpu.SemaphoreType.DMA,
        ],
    )
    def depad_pack(table_hbm, out_hbm, a_v, b_v, si0, si1, so0, so1):
        wid = lax.axis_index("s") * _NC + lax.axis_index("c")
        sin = (si0, si1)
        sout = (so0, so1)

        def chunk_of(k):
            return wid + _NW * k

        def fire_in(k, p):
            c = chunk_of(k)

            @pl.when(c < _NCH)
            def _():
                pltpu.async_copy(table_hbm.at[pl.ds(c * _CH, _CH)],
                                 a_v.at[p], sin[p])

        def wait_in(k, p):
            c = chunk_of(k)

            @pl.when(c < _NCH)
            def _():
                pltpu.make_async_copy(table_hbm.at[pl.ds(c * _CH, _CH)],
                                      a_v.at[p], sin[p]).wait()

        def fire_out(k, p):
            c = chunk_of(k)

            @pl.when(c < _NCH)
            def _():
                pltpu.async_copy(b_v.at[p],
                                 out_hbm.at[pl.ds(c * (_CH // 2), _CH // 2)],
                                 sout[p])

        def wait_out(k, p):
            c = chunk_of(k)

            @pl.when(c < _NCH)
            def _():
                pltpu.make_async_copy(
                    b_v.at[p],
                    out_hbm.at[pl.ds(c * (_CH // 2), _CH // 2)],
                    sout[p]).wait()

        def repack(k, p):
            c = chunk_of(k)

            @pl.when(c < _NCH)
            def _():
                def body(j, carry):
                    for q in range(4):
                        b_v[p, j, pl.ds(q * 16, 16)] = (
                            a_v[p, 2 * j, pl.ds(q * 16, 16)])
                        b_v[p, j, pl.ds(_D + q * 16, 16)] = (
                            a_v[p, 2 * j + 1, pl.ds(q * 16, 16)])
                    return carry

                lax.fori_loop(0, _CH // 2, body, 0, unroll=2)

        fire_in(0, 0)
        fire_in(1, 1)

        def step(kk, carry):
            for p in (0, 1):
                k = 2 * kk + p
                wait_in(k, p)

                @pl.when(k >= 2)
                def _():
                    wait_out(k - 2, p)

                repack(k, p)
                fire_out(k, p)
                fire_in(k + 2, p)
            return carry

        lax.fori_loop(0, _KMAX // 2, step, 0)
        wait_out(_KMAX - 2, 0)
        wait_out(_KMAX - 1, 1)

    return depad_pack


_depad_pack = _make_depad_pack()


def _classifier_kernel(p_ref, w_ref, b_ref, o_ref):
    o_ref[...] = (
        jnp.dot(p_ref[...], w_ref[...], preferred_element_type=jnp.float32)
        + b_ref[...]
    )


def _classifier(pooled, W, b2d):
    return pl.pallas_call(
        _classifier_kernel,
        out_shape=jax.ShapeDtypeStruct((_B, _C), jnp.float32),
    )(pooled, W, b2d)


@jax.jit
def kernel(x, table, W, b):
    table2 = _depad_pack(table)
    table3 = jnp.reshape(table2, (_V, _D))
    pooled = _gather_pool(x.astype(jnp.int32), table3)
    return _classifier(pooled, W, b.reshape(1, _C))


# speedup vs baseline: 1.3272x; 1.3272x over previous
import functools

import jax
import jax.numpy as jnp
from jax import lax
from jax.experimental import pallas as pl
from jax.experimental.pallas import tpu as pltpu
from jax.experimental.pallas import tpu_sc as plsc

_V = 1000000
_B = 4096
_L = 200
_D = 64
_C = 20
_NC = 2
_NS = 16
_NW = _NC * _NS
_SPW = _B // _NW
_H0 = 96
_H1 = _L - _H0


def _make_gather_pool():
    mesh = plsc.VectorSubcoreMesh(
        core_axis_name="c", subcore_axis_name="s",
        num_cores=_NC, num_subcores=_NS,
    )

    @functools.partial(
        pl.kernel,
        out_type=jax.ShapeDtypeStruct((_B, _D), jnp.float32),
        mesh=mesh,
        name="gather_pool",
        compiler_params=pltpu.CompilerParams(use_tc_tiling_on_sc=False),
        scratch_types=[
            pltpu.VMEM((_SPW, _L), jnp.int32),
            pltpu.VMEM((4, _L, _D), jnp.float32),
            pltpu.VMEM((_SPW, _D), jnp.float32),
            pltpu.SemaphoreType.DMA,
            pltpu.SemaphoreType.DMA,
            pltpu.SemaphoreType.DMA,
            pltpu.SemaphoreType.DMA,
        ],
    )
    def gather_pool(x_hbm, table_hbm, out_hbm,
                    idx_v, rows_v, pooled_v, s0, s1, s2, s3):
        wid = lax.axis_index("s") * _NC + lax.axis_index("c")
        sems = (s0, s1, s2, s3)
        pltpu.sync_copy(x_hbm.at[pl.ds(wid * _SPW, _SPW)], idx_v)

        def fire(i, p):
            @pl.when(i < _SPW)
            def _():
                pltpu.async_copy(table_hbm.at[idx_v.at[i, pl.ds(0, _H0)]],
                                 rows_v.at[p, pl.ds(0, _H0)], sems[p])
                pltpu.async_copy(table_hbm.at[idx_v.at[i, pl.ds(_H0, _H1)]],
                                 rows_v.at[p, pl.ds(_H0, _H1)], sems[p])

        def wait(i, p):
            pltpu.make_async_copy(table_hbm.at[idx_v.at[i, pl.ds(0, _H0)]],
                                  rows_v.at[p, pl.ds(0, _H0)], sems[p]).wait()
            pltpu.make_async_copy(table_hbm.at[idx_v.at[i, pl.ds(_H0, _H1)]],
                                  rows_v.at[p, pl.ds(_H0, _H1)], sems[p]).wait()

        def consume(i, p):
            def body(l, accs):
                return tuple(a + rows_v[p, l, pl.ds(q * 16, 16)]
                             for q, a in enumerate(accs))
            accs = tuple(jnp.zeros((16,), jnp.float32) for _ in range(4))
            accs = lax.fori_loop(0, _L, body, accs, unroll=4)
            for q in range(4):
                pooled_v[i, pl.ds(q * 16, 16)] = accs[q]

        for p in range(4):
            fire(p, p)

        def step(k, carry):
            for p in range(4):
                i = 4 * k + p
                wait(i, p)
                consume(i, p)
                fire(i + 4, p)
            return carry

        lax.fori_loop(0, _SPW // 4, step, 0)
        pltpu.sync_copy(pooled_v, out_hbm.at[pl.ds(wid * _SPW, _SPW)])

    return gather_pool


_gather_pool = _make_gather_pool()


def _classifier_kernel(p_ref, w_ref, b_ref, o_ref):
    o_ref[...] = (
        jnp.dot(p_ref[...], w_ref[...], preferred_element_type=jnp.float32)
        + b_ref[...]
    )


def _classifier(pooled, W, b2d):
    return pl.pallas_call(
        _classifier_kernel,
        out_shape=jax.ShapeDtypeStruct((_B, _C), jnp.float32),
    )(pooled, W, b2d)


@jax.jit
def kernel(x, table, W, b):
    table2 = jnp.reshape(table, (_V // 2, 2 * _D))
    table2 = jax.lax.optimization_barrier(table2)
    table3 = jnp.reshape(table2, (_V, _D))
    pooled = _gather_pool(x.astype(jnp.int32), table3)
    return _classifier(pooled, W, b.reshape(1, _C))
